# baseline (speedup 1.0000x reference)
import functools

import jax
import jax.numpy as jnp
from jax import lax
from jax.experimental import pallas as pl
from jax.experimental.pallas import tpu as pltpu
from jax.experimental.pallas import tpu_sc as plsc

_PAD = 8
_D = 300
_NW = 32
_CHUNK = 128


def _sc_gather(tbl, idx, n_tok):
  per_w = n_tok // _NW
  n_chunk = per_w // _CHUNK
  mesh = plsc.VectorSubcoreMesh(core_axis_name="c", subcore_axis_name="s")

  @functools.partial(
      pl.kernel,
      out_type=jax.ShapeDtypeStruct((n_tok, _D), jnp.float32),
      mesh=mesh,
      scratch_types=[
          pltpu.VMEM((n_chunk, _CHUNK), jnp.int32),
          pltpu.VMEM((_CHUNK, _D), jnp.float32),
          pltpu.SemaphoreType.DMA,
      ],
      compiler_params=pltpu.CompilerParams(use_tc_tiling_on_sc=False),
  )
  def k(tbl_hbm, idx_hbm, out_hbm, idx_v, buf, sem):
    wid = lax.axis_index("s") * 2 + lax.axis_index("c")
    base = wid * per_w
    pltpu.sync_copy(idx_hbm.at[wid], idx_v)

    def body(j, carry):
      pltpu.async_copy(tbl_hbm.at[idx_v.at[j]], buf, sem).wait()
      pltpu.sync_copy(buf, out_hbm.at[pl.ds(base + j * _CHUNK, _CHUNK)])
      return carry

    lax.fori_loop(0, n_chunk, body, 0)

  return k(tbl, idx)


def kernel(inputs, emb_table):
  b, l = inputs.shape
  n_tok = b * l
  per_w = n_tok // _NW
  idx = inputs.reshape(_NW, per_w // _CHUNK, _CHUNK).astype(jnp.int32)
  tbl = emb_table.at[0].set(0.0).at[_PAD].set(0.0)
  out = _sc_gather(tbl, idx, n_tok)
  return out.reshape(b, l, _D)
